# baseline (speedup 1.0000x reference)
import jax
import jax.numpy as jnp
from jax.experimental import pallas as pl
from jax.experimental.pallas import tpu as pltpu

_N, _D = 500000, 128
_BN = 20000


def _loss_kernel(w_ref, ya_ref, yb2_ref, out_ref):
    i = pl.program_id(0)
    wv = w_ref[...]
    y = jnp.concatenate([ya_ref[...], yb2_ref[...]], axis=0)
    inv_nw2 = 1.0 / jnp.sum(wv * wv)
    wib = wv * inv_nw2

    c = jnp.sum(y * wib, axis=1, keepdims=True)
    res = y - c * wv
    d2 = jnp.sum(res * res, axis=1, keepdims=True)

    lane = jax.lax.broadcasted_iota(jnp.int32, (1, _D), 1)
    leaf = (lane * 125) // _D
    digits = (leaf // 25, (leaf // 5) % 5, leaf % 5)
    lo = (leaf * _D + 124) // 125
    hi = ((leaf + 1) * _D + 124) // 125
    wlane = jnp.where(hi - lo == 2, 0.5, 1.0)

    zz = jnp.broadcast_to(d2, (_BN, _D))
    h = _BN
    for dig in digits:
        h //= 5
        parts = [zz[h * k:h * (k + 1), :] for k in range(5)]
        m = None
        for k in range(5):
            mk = (dig == k).astype(jnp.float32)
            term = parts[k] * mk
            m = term if m is None else m + term
        zz = m

    x = 1.0 + zz
    z = x * x - 1.0
    sq = z * jax.lax.rsqrt(z + 1e-30)
    a = jnp.log(x + sq)
    aa = a * a * wlane
    col = jnp.sum(aa, axis=0, keepdims=True)
    part = jnp.sum(col, axis=1, keepdims=True)

    @pl.when(i == 0)
    def _():
        out_ref[...] = jnp.zeros_like(out_ref)

    out_ref[...] += part


@jax.jit
def kernel(w, y):
    w2 = w.reshape(1, _D)
    out = pl.pallas_call(
        _loss_kernel,
        out_shape=jax.ShapeDtypeStruct((1, 1), jnp.float32),
        grid=(_N // _BN,),
        in_specs=[
            pl.BlockSpec((1, _D), lambda i: (0, 0)),
            pl.BlockSpec((_BN // 2, _D), lambda i: (2 * i, 0)),
            pl.BlockSpec((_BN // 2, _D), lambda i: (2 * i + 1, 0)),
        ],
        out_specs=pl.BlockSpec((1, 1), lambda i: (0, 0)),
        compiler_params=pltpu.CompilerParams(
            dimension_semantics=("arbitrary",),
            vmem_limit_bytes=56 * 1024 * 1024,
        ),
        name="hyperbolic_lines_loss",
    )(w2, y, y)
    return out[0, 0]


# speedup vs baseline: 1.0126x; 1.0126x over previous
import jax
import jax.numpy as jnp
from jax.experimental import pallas as pl
from jax.experimental.pallas import tpu as pltpu

_N, _D = 500000, 128
_BN = 25000


def _loss_kernel(w_ref, y_ref, out_ref):
    i = pl.program_id(0)
    wv = w_ref[...]
    y = y_ref[...]
    inv_nw2 = 1.0 / jnp.sum(wv * wv)
    wib = wv * inv_nw2

    c = jnp.sum(y * wib, axis=1, keepdims=True)
    res = y - c * wv
    d2 = jnp.sum(res * res, axis=1, keepdims=True)

    lane = jax.lax.broadcasted_iota(jnp.int32, (1, _D), 1)
    leaf = (lane * 125) // _D
    digits = (leaf // 25, (leaf // 5) % 5, leaf % 5)
    lo = (leaf * _D + 124) // 125
    hi = ((leaf + 1) * _D + 124) // 125
    wlane = jnp.where(hi - lo == 2, 0.5, 1.0)

    zz = jnp.broadcast_to(d2, (_BN, _D))
    h = _BN
    for dig in digits:
        h //= 5
        parts = [zz[h * k:h * (k + 1), :] for k in range(5)]
        m = None
        for k in range(5):
            mk = (dig == k).astype(jnp.float32)
            term = parts[k] * mk
            m = term if m is None else m + term
        zz = m

    x = 1.0 + zz
    z = x * x - 1.0
    sq = z * jax.lax.rsqrt(z + 1e-30)
    a = jnp.log(x + sq)
    aa = a * a * wlane
    col = jnp.sum(aa, axis=0, keepdims=True)
    part = jnp.sum(col, axis=1, keepdims=True)

    @pl.when(i == 0)
    def _():
        out_ref[...] = jnp.zeros_like(out_ref)

    out_ref[...] += part


@jax.jit
def kernel(w, y):
    w2 = w.reshape(1, _D)
    out = pl.pallas_call(
        _loss_kernel,
        out_shape=jax.ShapeDtypeStruct((1, 1), jnp.float32),
        grid=(_N // _BN,),
        in_specs=[
            pl.BlockSpec((1, _D), lambda i: (0, 0)),
            pl.BlockSpec((_BN, _D), lambda i: (i, 0)),
        ],
        out_specs=pl.BlockSpec((1, 1), lambda i: (0, 0)),
        compiler_params=pltpu.CompilerParams(
            dimension_semantics=("arbitrary",),
            vmem_limit_bytes=56 * 1024 * 1024,
        ),
        name="hyperbolic_lines_loss",
    )(w2, y)
    return out[0, 0]


# speedup vs baseline: 1.5176x; 1.4987x over previous
import jax
import jax.numpy as jnp
from jax.experimental import pallas as pl
from jax.experimental.pallas import tpu as pltpu

_N, _D = 500000, 128
_BN = 25000


def _loss_kernel(w_ref, wcol_ref, y_ref, out_ref):
    i = pl.program_id(0)
    wv = w_ref[...]
    wcol = wcol_ref[...]
    y = y_ref[...]
    inv_nw2 = 1.0 / jnp.sum(wv * wv)

    rhs = jnp.broadcast_to(wcol * inv_nw2, (_D, 2 * _D))
    c_rep = jax.lax.dot_general(
        y, rhs, (((1,), (0,)), ((), ())),
        preferred_element_type=jnp.float32)
    c = c_rep[:, :_D]
    res = y - c * wv
    d2 = jnp.sum(res * res, axis=1, keepdims=True)

    lane = jax.lax.broadcasted_iota(jnp.int32, (1, _D), 1)
    leaf = (lane * 125) // _D
    digits = (leaf // 25, (leaf // 5) % 5, leaf % 5)
    lo = (leaf * _D + 124) // 125
    hi = ((leaf + 1) * _D + 124) // 125
    wlane = jnp.where(hi - lo == 2, 0.5, 1.0)

    zz = jnp.broadcast_to(d2, (_BN, _D))
    h = _BN
    for dig in digits:
        h //= 5
        parts = [zz[h * k:h * (k + 1), :] for k in range(5)]
        m = None
        for k in range(5):
            mk = (dig == k).astype(jnp.float32)
            term = parts[k] * mk
            m = term if m is None else m + term
        zz = m

    x = 1.0 + zz
    z = x * x - 1.0
    sq = z * jax.lax.rsqrt(z + 1e-30)
    a = jnp.log(x + sq)
    aa = a * a * wlane
    col = jnp.sum(aa, axis=0, keepdims=True)
    part = jnp.sum(col, axis=1, keepdims=True)

    @pl.when(i == 0)
    def _():
        out_ref[...] = jnp.zeros_like(out_ref)

    out_ref[...] += part


@jax.jit
def kernel(w, y):
    w2 = w.reshape(1, _D)
    wcol = w.reshape(_D, 1)
    out = pl.pallas_call(
        _loss_kernel,
        out_shape=jax.ShapeDtypeStruct((1, 1), jnp.float32),
        grid=(_N // _BN,),
        in_specs=[
            pl.BlockSpec((1, _D), lambda i: (0, 0)),
            pl.BlockSpec((_D, 1), lambda i: (0, 0)),
            pl.BlockSpec((_BN, _D), lambda i: (i, 0)),
        ],
        out_specs=pl.BlockSpec((1, 1), lambda i: (0, 0)),
        compiler_params=pltpu.CompilerParams(
            dimension_semantics=("arbitrary",),
            vmem_limit_bytes=56 * 1024 * 1024,
        ),
        name="hyperbolic_lines_loss",
    )(w2, wcol, y)
    return out[0, 0]
